# baseline (speedup 1.0000x reference)
import jax
import jax.numpy as jnp
from jax.experimental import pallas as pl

_B, _V = 128, 100000
_BV = 12800
_VP = 102400


def _mask_body(logits_ref, cap_ref, out_ref):
    i = pl.program_id(0)
    cs = cap_ref[pl.ds(i * _BV, _BV)]
    out_ref[...] = jnp.minimum(logits_ref[...], cs.reshape(_BV, 1))


@jax.jit
def _run(logits, forbidden_token_mask):
    cap = jnp.where(forbidden_token_mask, -jnp.inf, jnp.inf)
    cap = jnp.pad(cap.astype(logits.dtype), (0, _VP - _V),
                  constant_values=jnp.inf)
    lt = logits.T
    out = pl.pallas_call(
        _mask_body,
        grid=(pl.cdiv(_V, _BV),),
        in_specs=[
            pl.BlockSpec((_BV, _B), lambda i: (i, 0)),
            pl.BlockSpec((_VP,), lambda i: (0,)),
        ],
        out_specs=pl.BlockSpec((_BV, _B), lambda i: (i, 0)),
        out_shape=jax.ShapeDtypeStruct((_V, _B), logits.dtype),
    )(lt, cap)
    return out.T


def kernel(logits, generated_so_far, forbidden_token_mask):
    return _run(logits, forbidden_token_mask)


# speedup vs baseline: 1.0151x; 1.0151x over previous
import jax
import jax.numpy as jnp
from jax.experimental import pallas as pl

_B, _V = 128, 100000
_BV = 25600
_VP = 102400


def _mask_body(logits_ref, cap_ref, out_ref):
    i = pl.program_id(0)
    cs = cap_ref[pl.ds(i * _BV, _BV)]
    out_ref[...] = jnp.minimum(logits_ref[...], cs.reshape(_BV, 1))


@jax.jit
def _run(logits, forbidden_token_mask):
    cap = jnp.where(forbidden_token_mask, -jnp.inf, jnp.inf)
    cap = jnp.pad(cap.astype(logits.dtype), (0, _VP - _V),
                  constant_values=jnp.inf)
    lt = logits.T
    out = pl.pallas_call(
        _mask_body,
        grid=(pl.cdiv(_V, _BV),),
        in_specs=[
            pl.BlockSpec((_BV, _B), lambda i: (i, 0)),
            pl.BlockSpec((_VP,), lambda i: (0,)),
        ],
        out_specs=pl.BlockSpec((_BV, _B), lambda i: (i, 0)),
        out_shape=jax.ShapeDtypeStruct((_V, _B), logits.dtype),
    )(lt, cap)
    return out.T


def kernel(logits, generated_so_far, forbidden_token_mask):
    return _run(logits, forbidden_token_mask)
